# Initial kernel scaffold; baseline (speedup 1.0000x reference)
import jax
import jax.numpy as jnp
from jax.experimental import pallas as pl


def kernel(x):
    raise NotImplementedError("write your pallas kernel here")


# speedup vs baseline: 6.9296x; 6.9296x over previous
import functools

import jax
import jax.numpy as jnp
from jax import lax
from jax.experimental import pallas as pl
from jax.experimental.pallas import tpu as pltpu
from jax.experimental.pallas import tpu_sc as plsc

KTOP = 100
ROWS = 4096
COLS = 16384
LANES = 16
NV = COLS // LANES
NC = 2
NS = 16
NW = NC * NS
ROWS_PER_W = ROWS // NW

def _sortable(xv):
    iv = lax.bitcast_convert_type(xv, jnp.int32)
    return jnp.where(iv < 0, iv ^ 0x7FFFFFFF, iv)


def _scan_suffix(hist_v, start_vreg, k_rank, iota):
    zero = jnp.int32(0)

    def cond(st):
        i, cum, found, b, cg, cb = st
        return jnp.logical_and(found == 0, i >= 0)

    def body(st):
        i, cum, found, b, cg, cb = st
        v = hist_v[pl.ds(i * LANES, LANES)]
        sfx = lax.rev(plsc.cumsum(lax.rev(v, (0,))), (0,))
        tot = jnp.sum(v)
        anyhit = (cum + tot) >= k_rank
        hit = (cum + sfx) >= k_rank
        j = jnp.max(jnp.where(hit, iota, -1))
        vj = jnp.sum(jnp.where(iota == j, v, zero))
        sj = jnp.sum(jnp.where(iota == j, sfx, zero))
        b_new = jnp.where(anyhit, i * LANES + j, b)
        cg_new = jnp.where(anyhit, cum + sj - vj, cg)
        cb_new = jnp.where(anyhit, vj, cb)
        found_new = jnp.where(anyhit, jnp.int32(1), found)
        return (i - 1, cum + tot, found_new, b_new, cg_new, cb_new)

    st = lax.while_loop(
        cond, body, (start_vreg, zero, zero, zero, zero, zero))
    return st[3], st[4], st[5]


def _clear_hist(hist_v, nvregs):
    zeros16 = jnp.zeros((LANES,), jnp.int32)

    def clr(i, c):
        hist_v[pl.ds(i * LANES, LANES)] = zeros16
        return c

    lax.fori_loop(0, nvregs, clr, 0, unroll=4)


def _topk_body(x_hbm, out_hbm, row_v, hist_v):
    wid = lax.axis_index("s") * NC + lax.axis_index("c")
    iota = lax.iota(jnp.int32, LANES)
    ones16 = jnp.ones((LANES,), jnp.int32)
    kk = jnp.int32(KTOP)

    def per_row(r, carry):
        row = wid * ROWS_PER_W + r
        pltpu.sync_copy(x_hbm.at[row], row_v)

        _clear_hist(hist_v, 2048 // LANES)

        def p1(i, vmax):
            uv = _sortable(row_v[pl.ds(i * LANES, LANES)])
            b = (uv >> 21) + 1024
            plsc.addupdate_scatter(hist_v, [b], ones16)
            return jnp.maximum(vmax, b)

        bmaxv = lax.fori_loop(
            0, NV, p1, jnp.full((LANES,), jnp.int32(-2147483647)), unroll=4)
        bmax = jnp.max(bmaxv)
        b1, cgt1, c1 = _scan_suffix(hist_v, bmax // LANES, kk, iota)
        hh1 = b1 - 1024

        _clear_hist(hist_v, 2048 // LANES)

        def p2(i, vmax):
            uv = _sortable(row_v[pl.ds(i * LANES, LANES)])
            act = (uv >> 21) == hh1
            b = (uv >> 10) & 0x7FF
            plsc.addupdate_scatter(hist_v, [b], ones16, mask=act)
            return jnp.maximum(vmax, jnp.where(act, b, -1))

        bmax2v = lax.fori_loop(
            0, NV, p2, jnp.full((LANES,), jnp.int32(-1)), unroll=4)
        k2 = kk - cgt1
        b2, cgt2, c2 = _scan_suffix(hist_v, jnp.max(bmax2v) // LANES, k2, iota)
        pre22 = (hh1 << 11) | b2

        _clear_hist(hist_v, 1024 // LANES)

        def p3(i, vmax):
            uv = _sortable(row_v[pl.ds(i * LANES, LANES)])
            act = (uv >> 10) == pre22
            b = uv & 0x3FF
            plsc.addupdate_scatter(hist_v, [b], ones16, mask=act)
            return jnp.maximum(vmax, jnp.where(act, b, -1))

        bmax3v = lax.fori_loop(
            0, NV, p3, jnp.full((LANES,), jnp.int32(-1)), unroll=4)
        k3 = k2 - cgt2
        b3, cgt3, ceq = _scan_suffix(hist_v, jnp.max(bmax3v) // LANES, k3, iota)
        t = (pre22 << 10) | b3
        m = k3 - cgt3

        zf = jnp.float32(0.0)

        def out_simple(_):
            def ob(i, c):
                xv = row_v[pl.ds(i * LANES, LANES)]
                keep = _sortable(xv) >= t
                row_v[pl.ds(i * LANES, LANES)] = jnp.where(keep, xv, zf)
                return c

            lax.fori_loop(0, NV, ob, 0, unroll=4)
            return 0

        def out_ties(_):
            def ob(i, c):
                xv = row_v[pl.ds(i * LANES, LANES)]
                uv = _sortable(xv)
                eq = uv == t
                eqi = eq.astype(jnp.int32)
                pref = plsc.cumsum(eqi)
                keep = jnp.logical_or(
                    uv > t, jnp.logical_and(eq, (pref + c) <= m))
                row_v[pl.ds(i * LANES, LANES)] = jnp.where(keep, xv, zf)
                return c + jnp.sum(eqi)

            lax.fori_loop(0, NV, ob, jnp.int32(0))
            return 0

        lax.cond(m == ceq, out_simple, out_ties, 0)

        pltpu.sync_copy(row_v, out_hbm.at[row])
        return carry

    lax.fori_loop(0, ROWS_PER_W, per_row, 0)


def kernel(x):
    mesh = plsc.VectorSubcoreMesh(
        core_axis_name="c", subcore_axis_name="s",
        num_cores=NC, num_subcores=NS)
    fn = functools.partial(
        pl.kernel,
        mesh=mesh,
        compiler_params=pltpu.CompilerParams(needs_layout_passes=False),
        out_type=jax.ShapeDtypeStruct((ROWS, COLS), jnp.float32),
        scratch_types=[
            pltpu.VMEM((COLS,), jnp.float32),
            pltpu.VMEM((2048,), jnp.int32),
        ],
    )(_topk_body)
    return fn(x)


# speedup vs baseline: 18.8512x; 2.7204x over previous
import functools

import jax
import jax.numpy as jnp
from jax import lax
from jax.experimental import pallas as pl
from jax.experimental.pallas import tpu as pltpu
from jax.experimental.pallas import tpu_sc as plsc

KTOP = 100
ROWS = 4096
COLS = 16384
LANES = 16
NV = COLS // LANES
NC = 2
NS = 16
NW = NC * NS
ROWS_PER_W = ROWS // NW

def _sortable(xv):
    iv = lax.bitcast_convert_type(xv, jnp.int32)
    return jnp.where(iv < 0, iv ^ 0x7FFFFFFF, iv)


def _scan_suffix(hist_v, start_vreg, k_rank, iota):
    zero = jnp.int32(0)

    def cond(st):
        i, cum, found, b, cg, cb = st
        return jnp.logical_and(found == 0, i >= 0)

    def body(st):
        i, cum, found, b, cg, cb = st
        v = hist_v[pl.ds(i * LANES, LANES)]
        sfx = lax.rev(plsc.cumsum(lax.rev(v, (0,))), (0,))
        tot = jnp.sum(v)
        anyhit = (cum + tot) >= k_rank
        hit = (cum + sfx) >= k_rank
        j = jnp.max(jnp.where(hit, iota, -1))
        vj = jnp.sum(jnp.where(iota == j, v, zero))
        sj = jnp.sum(jnp.where(iota == j, sfx, zero))
        b_new = jnp.where(anyhit, i * LANES + j, b)
        cg_new = jnp.where(anyhit, cum + sj - vj, cg)
        cb_new = jnp.where(anyhit, vj, cb)
        found_new = jnp.where(anyhit, jnp.int32(1), found)
        return (i - 1, cum + tot, found_new, b_new, cg_new, cb_new)

    st = lax.while_loop(
        cond, body, (start_vreg, zero, zero, zero, zero, zero))
    return st[3], st[4], st[5]


def _clear_hist(hist_v, nvregs):
    zeros16 = jnp.zeros((LANES,), jnp.int32)

    @plsc.parallel_loop(0, nvregs, unroll=4)
    def _(i):
        hist_v[pl.ds(i * LANES, LANES)] = zeros16


def _topk_body(x_hbm, out_hbm, row_v, hist_v):
    wid = lax.axis_index("s") * NC + lax.axis_index("c")
    iota = lax.iota(jnp.int32, LANES)
    ones16 = jnp.ones((LANES,), jnp.int32)
    kk = jnp.int32(KTOP)

    def per_row(r, carry):
        row = wid * ROWS_PER_W + r
        pltpu.sync_copy(x_hbm.at[row], row_v)

        _clear_hist(hist_v, 2048 // LANES)

        @plsc.parallel_loop(
            0, NV, unroll=8, carry=jnp.full((LANES,), -2147483647, jnp.int32))
        def bmaxv(i, vmax):
            uv = _sortable(row_v[pl.ds(i * LANES, LANES)])
            b = (uv >> 21) + 1024
            plsc.addupdate_scatter(hist_v, [b], ones16)
            return jnp.maximum(vmax, b)

        bmax = jnp.max(bmaxv)
        b1, cgt1, c1 = _scan_suffix(hist_v, bmax // LANES, kk, iota)
        hh1 = b1 - 1024

        _clear_hist(hist_v, 2048 // LANES)

        @plsc.parallel_loop(
            0, NV, unroll=8, carry=jnp.full((LANES,), -1, jnp.int32))
        def bmax2v(i, vmax):
            uv = _sortable(row_v[pl.ds(i * LANES, LANES)])
            act = (uv >> 21) == hh1
            b = (uv >> 10) & 0x7FF
            plsc.addupdate_scatter(hist_v, [b], ones16, mask=act)
            return jnp.maximum(vmax, jnp.where(act, b, -1))

        k2 = kk - cgt1
        b2, cgt2, c2 = _scan_suffix(hist_v, jnp.max(bmax2v) // LANES, k2, iota)
        pre22 = (hh1 << 11) | b2

        _clear_hist(hist_v, 1024 // LANES)

        @plsc.parallel_loop(
            0, NV, unroll=8, carry=jnp.full((LANES,), -1, jnp.int32))
        def bmax3v(i, vmax):
            uv = _sortable(row_v[pl.ds(i * LANES, LANES)])
            act = (uv >> 10) == pre22
            b = uv & 0x3FF
            plsc.addupdate_scatter(hist_v, [b], ones16, mask=act)
            return jnp.maximum(vmax, jnp.where(act, b, -1))

        k3 = k2 - cgt2
        b3, cgt3, ceq = _scan_suffix(hist_v, jnp.max(bmax3v) // LANES, k3, iota)
        t = (pre22 << 10) | b3
        m = k3 - cgt3

        zf = jnp.float32(0.0)

        def out_simple(_):
            @plsc.parallel_loop(0, NV, unroll=8)
            def _(i):
                xv = row_v[pl.ds(i * LANES, LANES)]
                keep = _sortable(xv) >= t
                row_v[pl.ds(i * LANES, LANES)] = jnp.where(keep, xv, zf)

            return 0

        def out_ties(_):
            def ob(i, c):
                xv = row_v[pl.ds(i * LANES, LANES)]
                uv = _sortable(xv)
                eq = uv == t
                eqi = eq.astype(jnp.int32)
                pref = plsc.cumsum(eqi)
                keep = jnp.logical_or(
                    uv > t, jnp.logical_and(eq, (pref + c) <= m))
                row_v[pl.ds(i * LANES, LANES)] = jnp.where(keep, xv, zf)
                return c + jnp.sum(eqi)

            lax.fori_loop(0, NV, ob, jnp.int32(0))
            return 0

        lax.cond(m == ceq, out_simple, out_ties, 0)

        pltpu.sync_copy(row_v, out_hbm.at[row])
        return carry

    lax.fori_loop(0, ROWS_PER_W, per_row, 0)


def kernel(x):
    mesh = plsc.VectorSubcoreMesh(
        core_axis_name="c", subcore_axis_name="s",
        num_cores=NC, num_subcores=NS)
    fn = functools.partial(
        pl.kernel,
        mesh=mesh,
        compiler_params=pltpu.CompilerParams(needs_layout_passes=False),
        out_type=jax.ShapeDtypeStruct((ROWS, COLS), jnp.float32),
        scratch_types=[
            pltpu.VMEM((COLS,), jnp.float32),
            pltpu.VMEM((2048,), jnp.int32),
        ],
    )(_topk_body)
    return fn(x)


# speedup vs baseline: 20.8973x; 1.1085x over previous
import functools

import jax
import jax.numpy as jnp
from jax import lax
from jax.experimental import pallas as pl
from jax.experimental.pallas import tpu as pltpu
from jax.experimental.pallas import tpu_sc as plsc

KTOP = 100
ROWS = 4096
COLS = 16384
LANES = 16
NV = COLS // LANES
NC = 2
NS = 16
NW = NC * NS
ROWS_PER_W = ROWS // NW


def _sortable(xv):
    iv = lax.bitcast_convert_type(xv, jnp.int32)
    return jnp.where(iv < 0, iv ^ 0x7FFFFFFF, iv)


def _scan_suffix(hist_v, start_vreg, k_rank, iota):
    zero = jnp.int32(0)

    def cond(st):
        i, cum, found, b, cg, cb = st
        return jnp.logical_and(found == 0, i >= 0)

    def body(st):
        i, cum, found, b, cg, cb = st
        v = hist_v[pl.ds(i * LANES, LANES)]
        sfx = lax.rev(plsc.cumsum(lax.rev(v, (0,))), (0,))
        tot = jnp.sum(v)
        anyhit = (cum + tot) >= k_rank
        hit = (cum + sfx) >= k_rank
        j = jnp.max(jnp.where(hit, iota, -1))
        vj = jnp.sum(jnp.where(iota == j, v, zero))
        sj = jnp.sum(jnp.where(iota == j, sfx, zero))
        b_new = jnp.where(anyhit, i * LANES + j, b)
        cg_new = jnp.where(anyhit, cum + sj - vj, cg)
        cb_new = jnp.where(anyhit, vj, cb)
        found_new = jnp.where(anyhit, jnp.int32(1), found)
        return (i - 1, cum + tot, found_new, b_new, cg_new, cb_new)

    st = lax.while_loop(
        cond, body, (start_vreg, zero, zero, zero, zero, zero))
    return st[3], st[4], st[5]


def _clear_hist(hist_v, nvregs):
    zeros16 = jnp.zeros((LANES,), jnp.int32)

    @plsc.parallel_loop(0, nvregs, unroll=4)
    def _(i):
        hist_v[pl.ds(i * LANES, LANES)] = zeros16


def _select_and_mask(row_v, hist_v, iota, ones16):
    kk = jnp.int32(KTOP)

    _clear_hist(hist_v, 2048 // LANES)

    @plsc.parallel_loop(
        0, NV, unroll=8, carry=jnp.full((LANES,), -2147483647, jnp.int32))
    def bmaxv(i, vmax):
        uv = _sortable(row_v[pl.ds(i * LANES, LANES)])
        b = (uv >> 21) + 1024
        plsc.addupdate_scatter(hist_v, [b], ones16)
        return jnp.maximum(vmax, b)

    bmax = jnp.max(bmaxv)
    b1, cgt1, c1 = _scan_suffix(hist_v, bmax // LANES, kk, iota)
    hh1 = b1 - 1024

    _clear_hist(hist_v, 2048 // LANES)

    @plsc.parallel_loop(
        0, NV, unroll=8, carry=jnp.full((LANES,), -1, jnp.int32))
    def bmax2v(i, vmax):
        uv = _sortable(row_v[pl.ds(i * LANES, LANES)])
        act = (uv >> 21) == hh1
        b = (uv >> 10) & 0x7FF
        plsc.addupdate_scatter(hist_v, [b], ones16, mask=act)
        return jnp.maximum(vmax, jnp.where(act, b, -1))

    k2 = kk - cgt1
    b2, cgt2, c2 = _scan_suffix(hist_v, jnp.max(bmax2v) // LANES, k2, iota)
    pre22 = (hh1 << 11) | b2

    _clear_hist(hist_v, 1024 // LANES)

    @plsc.parallel_loop(
        0, NV, unroll=8, carry=jnp.full((LANES,), -1, jnp.int32))
    def bmax3v(i, vmax):
        uv = _sortable(row_v[pl.ds(i * LANES, LANES)])
        act = (uv >> 10) == pre22
        b = uv & 0x3FF
        plsc.addupdate_scatter(hist_v, [b], ones16, mask=act)
        return jnp.maximum(vmax, jnp.where(act, b, -1))

    k3 = k2 - cgt2
    b3, cgt3, ceq = _scan_suffix(hist_v, jnp.max(bmax3v) // LANES, k3, iota)
    t = (pre22 << 10) | b3
    m = k3 - cgt3

    zf = jnp.float32(0.0)

    def out_simple(_):
        @plsc.parallel_loop(0, NV, unroll=8)
        def _(i):
            xv = row_v[pl.ds(i * LANES, LANES)]
            keep = _sortable(xv) >= t
            row_v[pl.ds(i * LANES, LANES)] = jnp.where(keep, xv, zf)

        return 0

    def out_ties(_):
        def ob(i, c):
            xv = row_v[pl.ds(i * LANES, LANES)]
            uv = _sortable(xv)
            eq = uv == t
            eqi = eq.astype(jnp.int32)
            pref = plsc.cumsum(eqi)
            keep = jnp.logical_or(
                uv > t, jnp.logical_and(eq, (pref + c) <= m))
            row_v[pl.ds(i * LANES, LANES)] = jnp.where(keep, xv, zf)
            return c + jnp.sum(eqi)

        lax.fori_loop(0, NV, ob, jnp.int32(0))
        return 0

    lax.cond(m == ceq, out_simple, out_ties, 0)


def _topk_body(x_hbm, out_hbm, row_a, row_b, hist_v,
               sem_ia, sem_ib, sem_oa, sem_ob):
    wid = lax.axis_index("s") * NC + lax.axis_index("c")
    base = wid * ROWS_PER_W
    iota = lax.iota(jnp.int32, LANES)
    ones16 = jnp.ones((LANES,), jnp.int32)
    npair = ROWS_PER_W // 2

    pltpu.async_copy(x_hbm.at[base], row_a, sem_ia)

    def per_pair(rr, carry):
        r0 = base + 2 * rr
        r1 = r0 + 1

        @pl.when(rr > 0)
        def _():
            pltpu.make_async_copy(row_b, out_hbm.at[r1 - 2], sem_ob).wait()

        pltpu.async_copy(x_hbm.at[r1], row_b, sem_ib)

        pltpu.make_async_copy(x_hbm.at[r0], row_a, sem_ia).wait()
        _select_and_mask(row_a, hist_v, iota, ones16)
        pltpu.async_copy(row_a, out_hbm.at[r0], sem_oa)

        pltpu.make_async_copy(x_hbm.at[r1], row_b, sem_ib).wait()
        _select_and_mask(row_b, hist_v, iota, ones16)
        pltpu.async_copy(row_b, out_hbm.at[r1], sem_ob)

        @pl.when(rr < npair - 1)
        def _():
            pltpu.make_async_copy(row_a, out_hbm.at[r0], sem_oa).wait()
            pltpu.async_copy(x_hbm.at[r0 + 2], row_a, sem_ia)

        return carry

    lax.fori_loop(0, npair, per_pair, 0)

    last = base + ROWS_PER_W - 1
    pltpu.make_async_copy(row_a, out_hbm.at[last - 1], sem_oa).wait()
    pltpu.make_async_copy(row_b, out_hbm.at[last], sem_ob).wait()


def kernel(x):
    mesh = plsc.VectorSubcoreMesh(
        core_axis_name="c", subcore_axis_name="s",
        num_cores=NC, num_subcores=NS)
    fn = functools.partial(
        pl.kernel,
        mesh=mesh,
        compiler_params=pltpu.CompilerParams(needs_layout_passes=False),
        out_type=jax.ShapeDtypeStruct((ROWS, COLS), jnp.float32),
        scratch_types=[
            pltpu.VMEM((COLS,), jnp.float32),
            pltpu.VMEM((COLS,), jnp.float32),
            pltpu.VMEM((2048,), jnp.int32),
            pltpu.SemaphoreType.DMA,
            pltpu.SemaphoreType.DMA,
            pltpu.SemaphoreType.DMA,
            pltpu.SemaphoreType.DMA,
        ],
    )(_topk_body)
    return fn(x)


# speedup vs baseline: 23.7673x; 1.1373x over previous
import functools

import jax
import jax.numpy as jnp
from jax import lax
from jax.experimental import pallas as pl
from jax.experimental.pallas import tpu as pltpu
from jax.experimental.pallas import tpu_sc as plsc

KTOP = 100
ROWS = 4096
COLS = 16384
LANES = 16
NV = COLS // LANES
NC = 2
NS = 16
NW = NC * NS
ROWS_PER_W = ROWS // NW
SAMPLE_STRIDE = 8
QLO = 28


def _sortable(xv):
    iv = lax.bitcast_convert_type(xv, jnp.int32)
    return jnp.where(iv < 0, iv ^ 0x7FFFFFFF, iv)


def _scan_suffix(hist_v, start_vreg, k_rank, iota):
    zero = jnp.int32(0)

    def cond(st):
        i, cum, found, b, cg, cb = st
        return jnp.logical_and(found == 0, i >= 0)

    def body(st):
        i, cum, found, b, cg, cb = st
        v = hist_v[pl.ds(i * LANES, LANES)]
        sfx = lax.rev(plsc.cumsum(lax.rev(v, (0,))), (0,))
        tot = jnp.sum(v)
        anyhit = (cum + tot) >= k_rank
        hit = (cum + sfx) >= k_rank
        j = jnp.max(jnp.where(hit, iota, -1))
        vj = jnp.sum(jnp.where(iota == j, v, zero))
        sj = jnp.sum(jnp.where(iota == j, sfx, zero))
        b_new = jnp.where(anyhit, i * LANES + j, b)
        cg_new = jnp.where(anyhit, cum + sj - vj, cg)
        cb_new = jnp.where(anyhit, vj, cb)
        found_new = jnp.where(anyhit, jnp.int32(1), found)
        return (i - 1, cum + tot, found_new, b_new, cg_new, cb_new)

    st = lax.while_loop(
        cond, body, (start_vreg, zero, zero, zero, zero, zero))
    return st[3], st[4], st[5]


def _clear_hist(hist_v, nvregs):
    zeros16 = jnp.zeros((LANES,), jnp.int32)

    @plsc.parallel_loop(0, nvregs, unroll=4)
    def _(i):
        hist_v[pl.ds(i * LANES, LANES)] = zeros16


def _radix_threshold(buf_v, nv, limit, k_rank, hist_v, iota, ones16,
                     is_float):
    def load_u(i):
        v = buf_v[pl.ds(i * LANES, LANES)]
        return _sortable(v) if is_float else v

    def lane_ok(i):
        if limit is None:
            return None
        return (i * LANES + iota) < limit

    def conj(a, b):
        return b if a is None else jnp.logical_and(a, b)

    def masked_add(mask, b):
        if mask is None:
            plsc.addupdate_scatter(hist_v, [b], ones16)
        else:
            plsc.addupdate_scatter(hist_v, [b], ones16, mask=mask)

    def masked_max(vmax, mask, b):
        if mask is None:
            return jnp.maximum(vmax, b)
        return jnp.maximum(vmax, jnp.where(mask, b, -1))

    _clear_hist(hist_v, 2048 // LANES)

    @plsc.parallel_loop(
        0, nv, unroll=8, carry=jnp.full((LANES,), -1, jnp.int32))
    def bmax1v(i, vmax):
        uv = load_u(i)
        ok = lane_ok(i)
        b = (uv >> 21) + 1024
        masked_add(ok, b)
        return masked_max(vmax, ok, b)

    b1, cgt1, c1 = _scan_suffix(hist_v, jnp.max(bmax1v) // LANES, k_rank, iota)
    hh1 = b1 - 1024

    _clear_hist(hist_v, 2048 // LANES)

    @plsc.parallel_loop(
        0, nv, unroll=8, carry=jnp.full((LANES,), -1, jnp.int32))
    def bmax2v(i, vmax):
        uv = load_u(i)
        act = conj(lane_ok(i), (uv >> 21) == hh1)
        b = (uv >> 10) & 0x7FF
        masked_add(act, b)
        return masked_max(vmax, act, b)

    k2 = k_rank - cgt1
    b2, cgt2, c2 = _scan_suffix(hist_v, jnp.max(bmax2v) // LANES, k2, iota)
    pre22 = (hh1 << 11) | b2

    _clear_hist(hist_v, 1024 // LANES)

    @plsc.parallel_loop(
        0, nv, unroll=8, carry=jnp.full((LANES,), -1, jnp.int32))
    def bmax3v(i, vmax):
        uv = load_u(i)
        act = conj(lane_ok(i), (uv >> 10) == pre22)
        b = uv & 0x3FF
        masked_add(act, b)
        return masked_max(vmax, act, b)

    k3 = k2 - cgt2
    b3, cgt3, ceq = _scan_suffix(hist_v, jnp.max(bmax3v) // LANES, k3, iota)
    t = (pre22 << 10) | b3
    m = k3 - cgt3
    return t, m, ceq


def _write_output(row_v, t, m, ceq):
    zf = jnp.float32(0.0)

    def out_simple(_):
        @plsc.parallel_loop(0, NV, unroll=8)
        def _(i):
            xv = row_v[pl.ds(i * LANES, LANES)]
            keep = _sortable(xv) >= t
            row_v[pl.ds(i * LANES, LANES)] = jnp.where(keep, xv, zf)

        return 0

    def out_ties(_):
        def ob(i, c):
            xv = row_v[pl.ds(i * LANES, LANES)]
            uv = _sortable(xv)
            eq = uv == t
            eqi = eq.astype(jnp.int32)
            pref = plsc.cumsum(eqi)
            keep = jnp.logical_or(
                uv > t, jnp.logical_and(eq, (pref + c) <= m))
            row_v[pl.ds(i * LANES, LANES)] = jnp.where(keep, xv, zf)
            return c + jnp.sum(eqi)

        lax.fori_loop(0, NV, ob, jnp.int32(0))
        return 0

    lax.cond(m == ceq, out_simple, out_ties, 0)


def _select_and_mask(row_v, cand_v, hist_v, iota, ones16):
    kk = jnp.int32(KTOP)

    _clear_hist(hist_v, 2048 // LANES)

    @plsc.parallel_loop(
        0, NV // SAMPLE_STRIDE, unroll=8,
        carry=jnp.full((LANES,), -1, jnp.int32))
    def bmaxsv(i, vmax):
        uv = _sortable(row_v[pl.ds(i * SAMPLE_STRIDE * LANES, LANES)])
        b = (uv >> 21) + 1024
        plsc.addupdate_scatter(hist_v, [b], ones16)
        return jnp.maximum(vmax, b)

    bhi = jnp.max(bmaxsv)
    blo, _, _ = _scan_suffix(hist_v, bhi // LANES, jnp.int32(QLO), iota)
    hhi = bhi - 1024
    hlo = blo - 1024

    zero16 = jnp.zeros((LANES,), jnp.int32)
    one = jnp.int32(1)

    @plsc.parallel_loop(0, NV, unroll=8, carry=(zero16, zero16))
    def acc_c(i, st):
        acc, cvec = st
        uv = _sortable(row_v[pl.ds(i * LANES, LANES)])
        hb = uv >> 21
        above = hb > hhi
        inr = jnp.logical_and(hb >= hlo, hb <= hhi)
        inri = inr.astype(jnp.int32)
        pref = plsc.cumsum(inri)
        idx = cvec + pref - 1
        plsc.store_scatter(cand_v, [idx], uv, mask=inr)
        cnt = plsc.all_reduce_population_count(inr)
        return (acc + jnp.where(above, one, 0), cvec + cnt)

    accv, cvec = acc_c
    n_above = jnp.sum(accv)
    n_cand = jnp.max(cvec)
    valid = jnp.logical_and(n_above < kk, (n_above + n_cand) >= kk)

    def fast(_):
        nvc = (n_cand + (LANES - 1)) // LANES
        t, m, ceq = _radix_threshold(
            cand_v, nvc, n_cand, kk - n_above, hist_v, iota, ones16,
            is_float=False)
        return t, m, ceq

    def classic(_):
        return _radix_threshold(
            row_v, NV, None, kk, hist_v, iota, ones16, is_float=True)

    t, m, ceq = lax.cond(valid, fast, classic, 0)
    _write_output(row_v, t, m, ceq)


def _topk_body(x_hbm, out_hbm, row_a, row_b, cand_v, hist_v,
               sem_ia, sem_ib, sem_oa, sem_ob):
    wid = lax.axis_index("s") * NC + lax.axis_index("c")
    base = wid * ROWS_PER_W
    iota = lax.iota(jnp.int32, LANES)
    ones16 = jnp.ones((LANES,), jnp.int32)
    npair = ROWS_PER_W // 2

    pltpu.async_copy(x_hbm.at[base], row_a, sem_ia)

    def per_pair(rr, carry):
        r0 = base + 2 * rr
        r1 = r0 + 1

        @pl.when(rr > 0)
        def _():
            pltpu.make_async_copy(row_b, out_hbm.at[r1 - 2], sem_ob).wait()

        pltpu.async_copy(x_hbm.at[r1], row_b, sem_ib)

        pltpu.make_async_copy(x_hbm.at[r0], row_a, sem_ia).wait()
        _select_and_mask(row_a, cand_v, hist_v, iota, ones16)
        pltpu.async_copy(row_a, out_hbm.at[r0], sem_oa)

        pltpu.make_async_copy(x_hbm.at[r1], row_b, sem_ib).wait()
        _select_and_mask(row_b, cand_v, hist_v, iota, ones16)
        pltpu.async_copy(row_b, out_hbm.at[r1], sem_ob)

        @pl.when(rr < npair - 1)
        def _():
            pltpu.make_async_copy(row_a, out_hbm.at[r0], sem_oa).wait()
            pltpu.async_copy(x_hbm.at[r0 + 2], row_a, sem_ia)

        return carry

    lax.fori_loop(0, npair, per_pair, 0)

    last = base + ROWS_PER_W - 1
    pltpu.make_async_copy(row_a, out_hbm.at[last - 1], sem_oa).wait()
    pltpu.make_async_copy(row_b, out_hbm.at[last], sem_ob).wait()


def kernel(x):
    mesh = plsc.VectorSubcoreMesh(
        core_axis_name="c", subcore_axis_name="s",
        num_cores=NC, num_subcores=NS)
    fn = functools.partial(
        pl.kernel,
        mesh=mesh,
        compiler_params=pltpu.CompilerParams(needs_layout_passes=False),
        out_type=jax.ShapeDtypeStruct((ROWS, COLS), jnp.float32),
        scratch_types=[
            pltpu.VMEM((COLS,), jnp.float32),
            pltpu.VMEM((COLS,), jnp.float32),
            pltpu.VMEM((COLS,), jnp.int32),
            pltpu.VMEM((2048,), jnp.int32),
            pltpu.SemaphoreType.DMA,
            pltpu.SemaphoreType.DMA,
            pltpu.SemaphoreType.DMA,
            pltpu.SemaphoreType.DMA,
        ],
    )(_topk_body)
    return fn(x)
